# initial kernel scaffold (unmeasured)
import jax
import jax.numpy as jnp
from jax import lax
from jax.experimental import pallas as pl
from jax.experimental.pallas import tpu as pltpu

NZ = 4
CH = 128

_MESH = pl.DeviceIdType.MESH


def _z_barrier(mz, mx, my):
    bar = pltpu.get_barrier_semaphore()
    for dz in range(NZ):

        @pl.when(dz != mz)
        def _():
            pl.semaphore_signal(
                bar, inc=1, device_id=(mx, my, dz), device_id_type=_MESH
            )

    pl.semaphore_wait(bar, NZ - 1)


def _counts_allgather(counts_row):

    def body(c_ref, out_ref, ssem, rsem):
        mx = lax.axis_index("x")
        my = lax.axis_index("y")
        mz = lax.axis_index("z")
        _z_barrier(mz, mx, my)

        out_ref[pl.ds(mz, 1), :] = c_ref[...]
        for dz in range(NZ):

            @pl.when(dz != mz)
            def _():
                rd = pltpu.make_async_remote_copy(
                    src_ref=c_ref,
                    dst_ref=out_ref.at[pl.ds(mz, 1)],
                    send_sem=ssem,
                    recv_sem=rsem,
                    device_id=(mx, my, dz),
                    device_id_type=_MESH,
                )
                rd.start()

        waiter = pltpu.make_async_remote_copy(
            src_ref=c_ref,
            dst_ref=out_ref.at[pl.ds(0, 1)],
            send_sem=ssem,
            recv_sem=rsem,
            device_id=(mx, my, mz),
            device_id_type=_MESH,
        )
        for _ in range(NZ - 1):
            waiter.wait_send()
        for _ in range(NZ - 1):
            waiter.wait_recv()

    return pl.pallas_call(
        body,
        out_shape=jax.ShapeDtypeStruct((NZ, NZ), jnp.int32),
        in_specs=[pl.BlockSpec(memory_space=pltpu.VMEM)],
        out_specs=pl.BlockSpec(memory_space=pltpu.VMEM),
        scratch_shapes=[pltpu.SemaphoreType.DMA, pltpu.SemaphoreType.DMA],
        compiler_params=pltpu.CompilerParams(collective_id=0),
    )(counts_row)


def _a2av(x_sorted, cnt_mat, src_off):
    m, n = x_sorted.shape

    def body(x_ref, cnt_ref, soff_ref, out_ref, ssem, rsem):
        mx = lax.axis_index("x")
        my = lax.axis_index("y")
        mz = lax.axis_index("z")
        _z_barrier(mz, mx, my)

        def nchunks(c):
            return (c + CH - 1) // CH

        def dst_base(col):
            acc = jnp.int32(0)
            for z in range(NZ):
                acc = acc + jnp.where(jnp.int32(z) < mz, cnt_ref[z, col], 0)
            return acc

        c_me = cnt_ref[mz, mz]
        s0 = soff_ref[mz]
        d0 = dst_base(mz)

        def lbody(i, carry):
            s = jnp.minimum(i * CH, c_me - CH)
            out_ref[pl.ds(d0 + s, CH), :] = x_ref[pl.ds(s0 + s, CH), :]
            return carry

        lax.fori_loop(0, nchunks(c_me), lbody, 0)

        nsend = jnp.int32(0)
        for d in range(NZ):
            cd = cnt_ref[mz, d]
            sb = soff_ref[d]
            db = dst_base(d)
            nd = jnp.where(mz == jnp.int32(d), jnp.int32(0), nchunks(cd))

            def sbody(i, carry, cd=cd, sb=sb, db=db, d=d):
                s = jnp.minimum(i * CH, cd - CH)
                rd = pltpu.make_async_remote_copy(
                    src_ref=x_ref.at[pl.ds(sb + s, CH)],
                    dst_ref=out_ref.at[pl.ds(db + s, CH)],
                    send_sem=ssem,
                    recv_sem=rsem,
                    device_id=(mx, my, d),
                    device_id_type=_MESH,
                )
                rd.start()
                return carry

            lax.fori_loop(0, nd, sbody, 0)
            nsend = nsend + nd

        waiter = pltpu.make_async_remote_copy(
            src_ref=x_ref.at[pl.ds(0, CH)],
            dst_ref=out_ref.at[pl.ds(0, CH)],
            send_sem=ssem,
            recv_sem=rsem,
            device_id=(mx, my, mz),
            device_id_type=_MESH,
        )
        nrecv = jnp.int32(0)
        for z in range(NZ):
            nrecv = nrecv + jnp.where(
                jnp.int32(z) == mz, jnp.int32(0), nchunks(cnt_ref[z, mz])
            )

        def wrecv(i, carry):
            waiter.wait_recv()
            return carry

        def wsend(i, carry):
            waiter.wait_send()
            return carry

        lax.fori_loop(0, nrecv, wrecv, 0)
        lax.fori_loop(0, nsend, wsend, 0)

    return pl.pallas_call(
        body,
        out_shape=jax.ShapeDtypeStruct((m, n), x_sorted.dtype),
        in_specs=[
            pl.BlockSpec(memory_space=pltpu.VMEM),
            pl.BlockSpec(memory_space=pltpu.SMEM),
            pl.BlockSpec(memory_space=pltpu.SMEM),
        ],
        out_specs=pl.BlockSpec(memory_space=pltpu.VMEM),
        scratch_shapes=[pltpu.SemaphoreType.DMA, pltpu.SemaphoreType.DMA],
        compiler_params=pltpu.CompilerParams(collective_id=1),
    )(x_sorted, cnt_mat, src_off)


def kernel(x, dest):
    dest = dest.astype(jnp.int32)
    perm = jnp.argsort(dest, stable=True)
    x_sorted = jnp.take(x, perm, axis=0)
    counts = jnp.bincount(dest, length=NZ).astype(jnp.int32)
    src_off = jnp.concatenate(
        [jnp.zeros((1,), jnp.int32), jnp.cumsum(counts)[:-1].astype(jnp.int32)]
    )
    cnt_mat = _counts_allgather(counts.reshape(1, NZ))
    return _a2av(x_sorted, cnt_mat, src_off)


# baseline (device time: 787658 ns/iter reference)
import jax
import jax.numpy as jnp
from jax import lax
from jax.experimental import pallas as pl
from jax.experimental.pallas import tpu as pltpu

NZ = 4
CH = 128
PAD = 1536

_MESH = pl.DeviceIdType.MESH


def _z_barrier(mz, mx, my):
    bar = pltpu.get_barrier_semaphore()
    for dz in range(NZ):

        @pl.when(dz != mz)
        def _():
            pl.semaphore_signal(
                bar, inc=1, device_id=(mx, my, dz), device_id_type=_MESH
            )

    pl.semaphore_wait(bar, NZ - 1)


def _counts_allgather(counts_row):

    def body(c_ref, out_ref, ssem, rsem):
        mx = lax.axis_index("x")
        my = lax.axis_index("y")
        mz = lax.axis_index("z")
        _z_barrier(mz, mx, my)

        out_ref[mz] = c_ref[...]
        for dz in range(NZ):

            @pl.when(dz != mz)
            def _():
                rd = pltpu.make_async_remote_copy(
                    src_ref=c_ref,
                    dst_ref=out_ref.at[mz],
                    send_sem=ssem,
                    recv_sem=rsem,
                    device_id=(mx, my, dz),
                    device_id_type=_MESH,
                )
                rd.start()

        waiter = pltpu.make_async_remote_copy(
            src_ref=c_ref,
            dst_ref=out_ref.at[0],
            send_sem=ssem,
            recv_sem=rsem,
            device_id=(mx, my, mz),
            device_id_type=_MESH,
        )
        for _ in range(NZ - 1):
            waiter.wait_send()
        for _ in range(NZ - 1):
            waiter.wait_recv()

    return pl.pallas_call(
        body,
        out_shape=jax.ShapeDtypeStruct((NZ, 1, 128), jnp.int32),
        in_specs=[pl.BlockSpec(memory_space=pltpu.VMEM)],
        out_specs=pl.BlockSpec(memory_space=pltpu.VMEM),
        scratch_shapes=[pltpu.SemaphoreType.DMA, pltpu.SemaphoreType.DMA],
        compiler_params=pltpu.CompilerParams(collective_id=0),
    )(counts_row)


def _aln8(v):
    return pl.multiple_of((v // 8) * 8, 8)


def _a2av_staged(x_sorted, cnt_mat, src_off):
    m, n = x_sorted.shape

    def body(x_ref, cnt_ref, soff_ref, stg_ref, ssem, rsem):
        mx = lax.axis_index("x")
        my = lax.axis_index("y")
        mz = lax.axis_index("z")
        _z_barrier(mz, mx, my)

        def expanded_len(start, count):
            return ((start + count + 7) // 8) * 8 - (start // 8) * 8

        def nchunks(ln):
            return (ln + CH - 1) // CH

        nsend = jnp.int32(0)
        for d in range(NZ):
            cd = cnt_ref[mz, d]
            sb = soff_ref[d]
            sb8 = _aln8(sb)
            ln = expanded_len(sb, cd)
            nd = nchunks(ln)

            def chunk_start(i, ln=ln):
                return pl.multiple_of(jnp.minimum(i * CH, ln - CH), 8)

            @pl.when(mz == jnp.int32(d))
            def _():
                def lbody(i, carry):
                    s = chunk_start(i)
                    stg_ref[mz, pl.ds(s, CH), :] = x_ref[pl.ds(sb8 + s, CH), :]
                    return carry

                lax.fori_loop(0, nd, lbody, 0)

            @pl.when(mz != jnp.int32(d))
            def _():
                def sbody(i, carry):
                    s = chunk_start(i)
                    rd = pltpu.make_async_remote_copy(
                        src_ref=x_ref.at[pl.ds(sb8 + s, CH)],
                        dst_ref=stg_ref.at[mz, pl.ds(s, CH)],
                        send_sem=ssem,
                        recv_sem=rsem,
                        device_id=(mx, my, d),
                        device_id_type=_MESH,
                    )
                    rd.start()
                    return carry

                lax.fori_loop(0, nd, sbody, 0)

            nsend = nsend + jnp.where(mz == jnp.int32(d), jnp.int32(0), nd)

        nrecv = jnp.int32(0)
        for z in range(NZ):
            s_z = jnp.int32(0)
            for dprime in range(NZ):
                s_z = s_z + jnp.where(
                    jnp.int32(dprime) < mz, cnt_ref[z, dprime], 0
                )
            ln_z = expanded_len(s_z, cnt_ref[z, mz])
            nrecv = nrecv + jnp.where(
                jnp.int32(z) == mz, jnp.int32(0), nchunks(ln_z)
            )

        waiter = pltpu.make_async_remote_copy(
            src_ref=x_ref.at[pl.ds(0, CH)],
            dst_ref=stg_ref.at[0, pl.ds(0, CH)],
            send_sem=ssem,
            recv_sem=rsem,
            device_id=(mx, my, mz),
            device_id_type=_MESH,
        )

        def wrecv(i, carry):
            waiter.wait_recv()
            return carry

        def wsend(i, carry):
            waiter.wait_send()
            return carry

        lax.fori_loop(0, nrecv, wrecv, 0)
        lax.fori_loop(0, nsend, wsend, 0)

    return pl.pallas_call(
        body,
        out_shape=jax.ShapeDtypeStruct((NZ, PAD, n), x_sorted.dtype),
        in_specs=[
            pl.BlockSpec(memory_space=pltpu.VMEM),
            pl.BlockSpec(memory_space=pltpu.SMEM),
            pl.BlockSpec(memory_space=pltpu.SMEM),
        ],
        out_specs=pl.BlockSpec(memory_space=pltpu.VMEM),
        scratch_shapes=[pltpu.SemaphoreType.DMA, pltpu.SemaphoreType.DMA],
        compiler_params=pltpu.CompilerParams(collective_id=1),
    )(x_sorted, cnt_mat, src_off)


def kernel(x, dest):
    dest = dest.astype(jnp.int32)
    perm = jnp.argsort(dest, stable=True)
    x_sorted = jnp.take(x, perm, axis=0)
    counts = jnp.bincount(dest, length=NZ).astype(jnp.int32)
    src_off = jnp.concatenate(
        [jnp.zeros((1,), jnp.int32), jnp.cumsum(counts)[:-1].astype(jnp.int32)]
    )

    counts_row = jnp.zeros((1, 128), jnp.int32).at[0, :NZ].set(counts)
    cnt_mat = _counts_allgather(counts_row)[:, 0, :NZ]

    staging = _a2av_staged(x_sorted, cnt_mat, src_off)

    me = lax.axis_index("z")
    cnt_me = jnp.take(cnt_mat, me, axis=1)
    csum = jnp.cumsum(cnt_me)
    dstbase = csum - cnt_me
    soff_all = jnp.cumsum(cnt_mat, axis=1) - cnt_mat
    s_z = jnp.take(soff_all, me, axis=1)
    a_z = s_z % 8

    r = jnp.arange(x.shape[0], dtype=jnp.int32)
    src = jnp.searchsorted(csum, r, side="right").astype(jnp.int32)
    idx = src * PAD + a_z[src] + (r - dstbase[src])
    return jnp.take(staging.reshape(NZ * PAD, x.shape[1]), idx, axis=0)


# device time: 242585 ns/iter; 3.2469x vs baseline; 3.2469x over previous
import jax
import jax.numpy as jnp
from jax import lax
from jax.experimental import pallas as pl
from jax.experimental.pallas import tpu as pltpu

NZ = 4

_MESH = pl.DeviceIdType.MESH


def _z_barrier(mz, mx, my):
    bar = pltpu.get_barrier_semaphore()
    for dz in range(NZ):

        @pl.when(dz != mz)
        def _():
            pl.semaphore_signal(
                bar, inc=1, device_id=(mx, my, dz), device_id_type=_MESH
            )

    pl.semaphore_wait(bar, NZ - 1)


def _counts_allgather(counts_row):

    def body(c_ref, out_ref, ssem, rsem):
        mx = lax.axis_index("x")
        my = lax.axis_index("y")
        mz = lax.axis_index("z")
        _z_barrier(mz, mx, my)

        out_ref[mz] = c_ref[...]
        for dz in range(NZ):

            @pl.when(dz != mz)
            def _():
                rd = pltpu.make_async_remote_copy(
                    src_ref=c_ref,
                    dst_ref=out_ref.at[mz],
                    send_sem=ssem,
                    recv_sem=rsem,
                    device_id=(mx, my, dz),
                    device_id_type=_MESH,
                )
                rd.start()

        waiter = pltpu.make_async_remote_copy(
            src_ref=c_ref,
            dst_ref=out_ref.at[0],
            send_sem=ssem,
            recv_sem=rsem,
            device_id=(mx, my, mz),
            device_id_type=_MESH,
        )
        for _ in range(NZ - 1):
            waiter.wait_send()
        for _ in range(NZ - 1):
            waiter.wait_recv()

    return pl.pallas_call(
        body,
        out_shape=jax.ShapeDtypeStruct((NZ, 1, 128), jnp.int32),
        in_specs=[pl.BlockSpec(memory_space=pltpu.VMEM)],
        out_specs=pl.BlockSpec(memory_space=pltpu.VMEM),
        scratch_shapes=[pltpu.SemaphoreType.DMA, pltpu.SemaphoreType.DMA],
        compiler_params=pltpu.CompilerParams(collective_id=0),
    )(counts_row)


def _a2av_rows(x3, dest, pos, cnt_mat):
    m = x3.shape[0]

    def body(x_ref, dest_ref, pos_ref, cnt_ref, out_ref, ssem, rsem, lsem):
        mx = lax.axis_index("x")
        my = lax.axis_index("y")
        mz = lax.axis_index("z")
        _z_barrier(mz, mx, my)

        def rowbody(i, carry):
            d_i = dest_ref[i]
            p_i = pos_ref[i]

            @pl.when(d_i != mz)
            def _():
                rd = pltpu.make_async_remote_copy(
                    src_ref=x_ref.at[i],
                    dst_ref=out_ref.at[p_i],
                    send_sem=ssem,
                    recv_sem=rsem,
                    device_id=(mx, my, d_i),
                    device_id_type=_MESH,
                )
                rd.start()

            @pl.when(d_i == mz)
            def _():
                cp = pltpu.make_async_copy(x_ref.at[i], out_ref.at[p_i], lsem)
                cp.start()

            return carry

        lax.fori_loop(0, m, rowbody, 0)

        nloc = cnt_ref[mz, mz]
        nrecv = jnp.int32(0)
        for z in range(NZ):
            nrecv = nrecv + jnp.where(
                jnp.int32(z) == mz, jnp.int32(0), cnt_ref[z, mz]
            )
        nsend = jnp.int32(m) - nloc

        lwaiter = pltpu.make_async_copy(x_ref.at[0], out_ref.at[0], lsem)
        rwaiter = pltpu.make_async_remote_copy(
            src_ref=x_ref.at[0],
            dst_ref=out_ref.at[0],
            send_sem=ssem,
            recv_sem=rsem,
            device_id=(mx, my, mz),
            device_id_type=_MESH,
        )

        def wloc(i, carry):
            lwaiter.wait()
            return carry

        def wrecv(i, carry):
            rwaiter.wait_recv()
            return carry

        def wsend(i, carry):
            rwaiter.wait_send()
            return carry

        lax.fori_loop(0, nloc, wloc, 0)
        lax.fori_loop(0, nrecv, wrecv, 0)
        lax.fori_loop(0, nsend, wsend, 0)

    return pl.pallas_call(
        body,
        out_shape=jax.ShapeDtypeStruct(x3.shape, x3.dtype),
        in_specs=[
            pl.BlockSpec(memory_space=pltpu.VMEM),
            pl.BlockSpec(memory_space=pltpu.SMEM),
            pl.BlockSpec(memory_space=pltpu.SMEM),
            pl.BlockSpec(memory_space=pltpu.SMEM),
        ],
        out_specs=pl.BlockSpec(memory_space=pltpu.VMEM),
        scratch_shapes=[
            pltpu.SemaphoreType.DMA,
            pltpu.SemaphoreType.DMA,
            pltpu.SemaphoreType.DMA,
        ],
        compiler_params=pltpu.CompilerParams(collective_id=1),
    )(x3, dest, pos, cnt_mat)


def kernel(x, dest):
    m, n = x.shape
    dest = dest.astype(jnp.int32)

    onehot = (dest[:, None] == jnp.arange(NZ, dtype=jnp.int32)[None, :]).astype(
        jnp.int32
    )
    counts = onehot.sum(axis=0)
    counts_row = jnp.zeros((1, 128), jnp.int32).at[0, :NZ].set(counts)
    cnt_mat = _counts_allgather(counts_row)[:, 0, :NZ]

    me = lax.axis_index("z")
    mask = jnp.arange(NZ, dtype=jnp.int32)[:, None] < me
    dstbase = jnp.sum(jnp.where(mask, cnt_mat, 0), axis=0)
    rank = jnp.cumsum(onehot, axis=0) - 1
    pos = jnp.sum(onehot * (rank + dstbase[None, :]), axis=1).astype(jnp.int32)

    x3 = x.reshape(m, 8, n // 8)
    out3 = _a2av_rows(x3, dest, pos, cnt_mat)
    return out3.reshape(m, n)
